# baseline (speedup 1.0000x reference)
import functools

import jax
import jax.numpy as jnp
from jax import lax
from jax.experimental import pallas as pl
from jax.experimental.pallas import tpu as pltpu
from jax.experimental.pallas import tpu_sc as plsc

_CHUNK = 128


@functools.lru_cache(maxsize=None)
def _build(batch, emb, n_cores, n_subcores):
    n_workers = n_cores * n_subcores
    b_per_w = batch // n_workers
    n_chunks = b_per_w // _CHUNK

    mesh = plsc.VectorSubcoreMesh(
        core_axis_name="c",
        subcore_axis_name="s",
        num_cores=n_cores,
        num_subcores=n_subcores,
    )

    row_block = jax.ShapeDtypeStruct(
        (n_workers, n_chunks, _CHUNK, emb), jnp.float32
    )

    @functools.partial(
        pl.kernel,
        mesh=mesh,
        out_type=(row_block, row_block),
        compiler_params=pltpu.CompilerParams(use_tc_tiling_on_sc=False),
        scratch_types=[
            pltpu.VMEM((n_chunks, _CHUNK), jnp.int32),
            pltpu.VMEM((n_chunks, _CHUNK), jnp.int32),
            pltpu.VMEM((n_chunks, _CHUNK, emb), jnp.float32),
            pltpu.VMEM((n_chunks, _CHUNK, emb), jnp.float32),
            pltpu.SemaphoreType.DMA,
            pltpu.SemaphoreType.DMA,
        ],
    )
    def gather_kernel(
        uid_hbm,
        iid_hbm,
        utab_hbm,
        itab_hbm,
        uout_hbm,
        iout_hbm,
        uidx_v,
        iidx_v,
        urows_v,
        irows_v,
        usem,
        isem,
    ):
        wid = lax.axis_index("s") * n_cores + lax.axis_index("c")

        pltpu.sync_copy(uid_hbm.at[wid], uidx_v)
        pltpu.sync_copy(iid_hbm.at[wid], iidx_v)

        ucopies = []
        icopies = []
        for c in range(n_chunks):
            ucopies.append(
                pltpu.async_copy(
                    utab_hbm.at[uidx_v.at[c]], urows_v.at[c], usem
                )
            )
            icopies.append(
                pltpu.async_copy(
                    itab_hbm.at[iidx_v.at[c]], irows_v.at[c], isem
                )
            )
        for cp in ucopies:
            cp.wait()
        pltpu.sync_copy(urows_v, uout_hbm.at[wid])
        for cp in icopies:
            cp.wait()
        pltpu.sync_copy(irows_v, iout_hbm.at[wid])

    return gather_kernel, n_workers, n_chunks


def kernel(user_id, item_id, user_table, item_table):
    batch = user_id.shape[0]
    emb = user_table.shape[1]
    info = plsc.get_sparse_core_info()
    fn, n_workers, n_chunks = _build(
        batch, emb, info.num_cores, info.num_subcores
    )

    uid = user_id.astype(jnp.int32).reshape(n_workers, n_chunks, _CHUNK)
    iid = item_id.astype(jnp.int32).reshape(n_workers, n_chunks, _CHUNK)

    u_rows, i_rows = fn(uid, iid, user_table, item_table)
    return (
        u_rows.reshape(batch, emb),
        i_rows.reshape(batch, emb),
    )


# speedup vs baseline: 1.0059x; 1.0059x over previous
import functools

import jax
import jax.numpy as jnp
from jax import lax
from jax.experimental import pallas as pl
from jax.experimental.pallas import tpu as pltpu
from jax.experimental.pallas import tpu_sc as plsc

_CHUNK = 128


@functools.lru_cache(maxsize=None)
def _build(batch, emb, n_cores, n_subcores):
    n_workers = n_cores * n_subcores
    b_per_w = batch // n_workers
    n_chunks = b_per_w // _CHUNK

    mesh = plsc.VectorSubcoreMesh(
        core_axis_name="c",
        subcore_axis_name="s",
        num_cores=n_cores,
        num_subcores=n_subcores,
    )

    @functools.partial(
        pl.kernel,
        mesh=mesh,
        out_type=jax.ShapeDtypeStruct(
            (n_workers, n_chunks, _CHUNK, emb), jnp.float32
        ),
        compiler_params=pltpu.CompilerParams(use_tc_tiling_on_sc=False),
        scratch_types=[
            pltpu.VMEM((n_chunks, _CHUNK), jnp.int32),
            pltpu.VMEM((n_chunks, _CHUNK, emb), jnp.float32),
            pltpu.SemaphoreType.DMA,
        ],
    )
    def gather_kernel(id_hbm, tab_hbm, out_hbm, idx_v, rows_v, sem):
        wid = lax.axis_index("s") * n_cores + lax.axis_index("c")

        pltpu.sync_copy(id_hbm.at[wid], idx_v)

        copies = []
        for c in range(n_chunks):
            copies.append(
                pltpu.async_copy(
                    tab_hbm.at[idx_v.at[c]], rows_v.at[c], sem
                )
            )
        for cp in copies:
            cp.wait()
        pltpu.sync_copy(rows_v, out_hbm.at[wid])

    return gather_kernel, n_workers, n_chunks


def kernel(user_id, item_id, user_table, item_table):
    batch = user_id.shape[0]
    emb = user_table.shape[1]
    info = plsc.get_sparse_core_info()
    fn, n_workers, n_chunks = _build(
        batch, emb, info.num_cores, info.num_subcores
    )

    uid = user_id.astype(jnp.int32).reshape(n_workers, n_chunks, _CHUNK)
    iid = item_id.astype(jnp.int32).reshape(n_workers, n_chunks, _CHUNK)

    u_rows = fn(uid, user_table)
    i_rows = fn(iid, item_table)
    return (
        u_rows.reshape(batch, emb),
        i_rows.reshape(batch, emb),
    )
